# baseline (speedup 1.0000x reference)
import functools

import jax
import jax.numpy as jnp
from jax import lax
from jax.experimental import pallas as pl
from jax.experimental.pallas import tpu as pltpu
from jax.experimental.pallas import tpu_sc as plsc

N = 10000
E = 160000
R = 8
NB = 4
D = 256

NC = 2
NS = 16
NW = NC * NS
EPW = E // NW
NSCAN = EPW // 16 + 1
NBKT = 64
BROWS = 157
ACC_R = 164
TRASH = 160
CAP = 256
CH = 16
CNTW = 64

NBLK = 10
BLK = N // NBLK

_SC_PARAMS = pltpu.CompilerParams(use_tc_tiling_on_sc=False,
                                  needs_layout_passes=False)


def _rel_weight(basis_ref, comp_ref, r):
    def b16(x):
        return x.astype(jnp.bfloat16).astype(jnp.float32)

    w = b16(comp_ref[r, 0]) * b16(basis_ref[0])
    for b in range(1, NB):
        w = w + b16(comp_ref[r, b]) * b16(basis_ref[b])
    return w


def _tc_first_body(x_ref, basis_ref, comp_ref, hall_ref):
    r = pl.program_id(1)
    w = _rel_weight(basis_ref, comp_ref, r)
    hall_ref[...] = jnp.dot(x_ref[...].astype(jnp.bfloat16),
                            w.astype(jnp.bfloat16),
                            preferred_element_type=jnp.float32)


def _tc_first(x, basis, comp):
    return pl.pallas_call(
        _tc_first_body,
        grid=(NBLK, R),
        in_specs=[
            pl.BlockSpec((BLK, D), lambda n, r: (n, 0)),
            pl.BlockSpec((NB, D, D), lambda n, r: (0, 0, 0)),
            pl.BlockSpec(memory_space=pltpu.SMEM),
        ],
        out_specs=pl.BlockSpec((BLK, D), lambda n, r: (n, r)),
        out_shape=jax.ShapeDtypeStruct((N, R * D), jnp.float32),
    )(x, basis, comp)


def _tc_mid_body(agg_ref, x_ref, lw_ref, b_ref, basis_ref, comp_ref,
                 h_ref, hall_ref, hs_ref):
    r = pl.program_id(1)

    @pl.when(r == 0)
    def _():
        t = agg_ref[...] + jnp.dot(x_ref[...].astype(jnp.bfloat16),
                                   lw_ref[...].astype(jnp.bfloat16),
                                   preferred_element_type=jnp.float32)
        t = jnp.maximum(t + b_ref[...], 0.0)
        hs_ref[...] = t
        h_ref[...] = t

    w = _rel_weight(basis_ref, comp_ref, r)
    hall_ref[...] = jnp.dot(hs_ref[...].astype(jnp.bfloat16),
                            w.astype(jnp.bfloat16),
                            preferred_element_type=jnp.float32)


def _tc_mid(agg, x, loop_w, bias, basis, comp):
    return pl.pallas_call(
        _tc_mid_body,
        grid=(NBLK, R),
        in_specs=[
            pl.BlockSpec((BLK, D), lambda n, r: (n, 0)),
            pl.BlockSpec((BLK, D), lambda n, r: (n, 0)),
            pl.BlockSpec((D, D), lambda n, r: (0, 0)),
            pl.BlockSpec((1, D), lambda n, r: (0, 0)),
            pl.BlockSpec((NB, D, D), lambda n, r: (0, 0, 0)),
            pl.BlockSpec(memory_space=pltpu.SMEM),
        ],
        out_specs=[
            pl.BlockSpec((BLK, D), lambda n, r: (n, 0)),
            pl.BlockSpec((BLK, D), lambda n, r: (n, r)),
        ],
        out_shape=[
            jax.ShapeDtypeStruct((N, D), jnp.float32),
            jax.ShapeDtypeStruct((N, R * D), jnp.float32),
        ],
        scratch_shapes=[pltpu.VMEM((BLK, D), jnp.float32)],
    )(agg, x, loop_w, bias, basis, comp)


def _tc_last_body(agg_ref, x_ref, lw_ref, b_ref, fcw_ref, fcb_ref, out_ref):
    t = agg_ref[...] + jnp.dot(x_ref[...].astype(jnp.bfloat16),
                               lw_ref[...].astype(jnp.bfloat16),
                               preferred_element_type=jnp.float32)
    h = jnp.maximum(t + b_ref[...], 0.0)
    t = jnp.dot(h.astype(jnp.bfloat16), fcw_ref[...].astype(jnp.bfloat16),
                preferred_element_type=jnp.float32)
    t = t + fcb_ref[...]
    m = jnp.max(t, axis=1, keepdims=True)
    e = jnp.exp(t - m)
    out_ref[...] = e / jnp.sum(e, axis=1, keepdims=True)


def _tc_last(agg, x, loop_w, bias, fc_w, fc_b):
    return pl.pallas_call(
        _tc_last_body,
        grid=(NBLK,),
        in_specs=[
            pl.BlockSpec((BLK, D), lambda n: (n, 0)),
            pl.BlockSpec((BLK, D), lambda n: (n, 0)),
            pl.BlockSpec((D, D), lambda n: (0, 0)),
            pl.BlockSpec((1, D), lambda n: (0, 0)),
            pl.BlockSpec((D, D), lambda n: (0, 0)),
            pl.BlockSpec((1, D), lambda n: (0, 0)),
        ],
        out_specs=pl.BlockSpec((BLK, D), lambda n: (n, 0)),
        out_shape=jax.ShapeDtypeStruct((N, D), jnp.float32),
    )(agg, x, loop_w, bias, fc_w, fc_b)


EBLK = 5000
CAPB = 4096
CH64 = 64
NBUF = 3
CNTS = 8


def _sc_mesh():
    return plsc.VectorSubcoreMesh(core_axis_name="c", subcore_axis_name="s",
                                  num_cores=NC, num_subcores=NS)


@functools.lru_cache(maxsize=None)
def _make_sc_route():
    return functools.partial(
        pl.kernel,
        mesh=_sc_mesh(),
        compiler_params=_SC_PARAMS,
        out_type=(jax.ShapeDtypeStruct((NBKT * CAPB,), jnp.int32),
                  jax.ShapeDtypeStruct((NBKT * CNTS,), jnp.int32)),
        scratch_types=[
            pltpu.VMEM((EBLK + 16,), jnp.int32),
            pltpu.VMEM((EBLK + 16,), jnp.int32),
            pltpu.VMEM((EBLK + 16,), jnp.int32),
            pltpu.VMEM((EBLK + 16,), jnp.int32),
            pltpu.VMEM((EBLK + 16,), jnp.int32),
            pltpu.VMEM((EBLK + 16,), jnp.int32),
            pltpu.VMEM((CAPB,), jnp.int32),
            pltpu.VMEM((CAPB,), jnp.int32),
            pltpu.VMEM((CAPB // 2,), jnp.int32),
            pltpu.VMEM((CAPB // 2,), jnp.int32),
            pltpu.VMEM((16,), jnp.int32),
        ],
    )(_sc_route_body)


def _sc_route_body(src_hbm, dst_hbm, et_hbm, packed_hbm, cnt_hbm,
                   src1_v, dst1_v, et1_v, src2_v, dst2_v, et2_v,
                   la_v, lb_v, la2_v, lb2_v, cb_v):
    cid = lax.axis_index("c")
    sid = lax.axis_index("s")
    w = cid * NS + sid
    bka = 2 * w
    iota = lax.iota(jnp.int32, 16)
    nvec = EBLK // 16
    tail = EBLK - nvec * 16
    nblk = E // EBLK // 2
    half2 = nblk * EBLK

    def stage(base, sv, dv, ev):
        pltpu.sync_copy(src_hbm.at[pl.ds(base, EBLK)], sv.at[pl.ds(0, EBLK)])
        pltpu.sync_copy(dst_hbm.at[pl.ds(base, EBLK)], dv.at[pl.ds(0, EBLK)])
        pltpu.sync_copy(et_hbm.at[pl.ds(base, EBLK)], ev.at[pl.ds(0, EBLK)])

    def step(sl, valid, sv, dv, ev, la, lb, cap, offa, offb):
        d = dv[sl]
        bk = d // BROWS
        pk = (sv[sl] * R + ev[sl]) * CAP + (d - bk * BROWS)
        ma = (bk == bka) & valid
        mb = (bk == bka + 1) & valid
        ca = plsc.all_reduce_population_count(ma)
        cb = plsc.all_reduce_population_count(mb)
        if getattr(ca, "ndim", 0):
            ca, cb = ca[0], cb[0]
        plsc.store_compressed(la.at[pl.ds(jnp.minimum(offa, cap - 16), 16)],
                              pk, mask=ma)
        plsc.store_compressed(lb.at[pl.ds(jnp.minimum(offb, cap - 16), 16)],
                              pk, mask=mb)
        return offa + ca, offb + cb

    def block_body(i, offs):
        offa, offb, offa2, offb2 = offs
        stage(i * EBLK, src1_v, dst1_v, et1_v)
        stage(half2 + i * EBLK, src2_v, dst2_v, et2_v)

        def vec_body(j, o):
            sl = pl.ds(j * 16, 16)
            oa, ob = step(sl, iota >= 0, src1_v, dst1_v, et1_v,
                          la_v, lb_v, CAPB, o[0], o[1])
            oa2, ob2 = step(sl, iota >= 0, src2_v, dst2_v, et2_v,
                            la2_v, lb2_v, CAPB // 2, o[2], o[3])
            return oa, ob, oa2, ob2

        offa, offb, offa2, offb2 = lax.fori_loop(
            0, nvec, vec_body, (offa, offb, offa2, offb2))
        sl = pl.ds(nvec * 16, 16)
        offa, offb = step(sl, iota < tail, src1_v, dst1_v, et1_v,
                          la_v, lb_v, CAPB, offa, offb)
        offa2, offb2 = step(sl, iota < tail, src2_v, dst2_v, et2_v,
                            la2_v, lb2_v, CAPB // 2, offa2, offb2)
        return offa, offb, offa2, offb2

    z = jnp.int32(0)
    offa, offb, offa2, offb2 = lax.fori_loop(0, nblk, block_body,
                                             (z, z, z, z))

    def concat(la, la2, offa, offa2):
        def mv(j, o):
            v = la2[pl.ds(j * 16, 16)]
            rest = offa2 - j * 16
            plsc.store_compressed(
                la.at[pl.ds(jnp.minimum(o, CAPB - 16), 16)], v,
                mask=iota < rest)
            return o + jnp.minimum(jnp.maximum(rest, 0), 16)

        nmv = (offa2 + 15) // 16
        return lax.fori_loop(0, nmv, mv, offa)

    offa = concat(la_v, la2_v, offa, offa2)
    offb = concat(lb_v, lb2_v, offb, offb2)

    sent = jnp.full((16,), TRASH, dtype=jnp.int32)
    for t in range(CH64 // 16):
        la_v[pl.ds(jnp.minimum(offa + t * 16, CAPB - 16), 16)] = sent
        lb_v[pl.ds(jnp.minimum(offb + t * 16, CAPB - 16), 16)] = sent
    cb_v[...] = jnp.where(iota == 0, offa, jnp.where(iota == 8, offb, 0))

    pltpu.sync_copy(la_v, packed_hbm.at[pl.ds(bka * CAPB, CAPB)])
    pltpu.sync_copy(lb_v, packed_hbm.at[pl.ds((bka + 1) * CAPB, CAPB)])
    pltpu.sync_copy(cb_v, cnt_hbm.at[pl.ds(bka * CNTS, 16)])


@functools.lru_cache(maxsize=None)
def _make_sc_scatter():
    return functools.partial(
        pl.kernel,
        mesh=_sc_mesh(),
        compiler_params=_SC_PARAMS,
        out_type=jax.ShapeDtypeStruct((N, D), jnp.float32),
        scratch_types=[
            pltpu.VMEM((NBKT * CNTS + 16,), jnp.int32),
            pltpu.VMEM((CAPB,), jnp.int32),
            pltpu.VMEM((NBUF, CH64), jnp.int32),
            pltpu.VMEM((NBUF, CH64, D), jnp.float32),
            pltpu.VMEM((ACC_R, D), jnp.float32),
        ] + [pltpu.SemaphoreType.DMA] * NBUF,
    )(_sc_scatter_body)


def _sc_scatter_body(hall_hbm, packed_hbm, cnt_hbm, out_hbm,
                     cnt_v, pk_v, gib, rowsb, acc, *gsem):
    cid = lax.axis_index("c")
    sid = lax.axis_index("s")
    w = cid * NS + sid

    pltpu.sync_copy(cnt_hbm, cnt_v.at[pl.ds(0, NBKT * CNTS)])
    zf = jnp.zeros((16,), jnp.float32)

    def bucket_pass(g, _):
        bkt = 2 * w + g

        def zero_body(i, _):
            for j in range(D // 16):
                acc[i, pl.ds(j * 16, 16)] = zf
            return 0

        lax.fori_loop(0, ACC_R, zero_body, 0)

        tot = cnt_v[pl.ds(bkt * CNTS, 16)][0]
        nch = (tot + CH64 - 1) // CH64
        base = bkt * CAPB
        pltpu.sync_copy(packed_hbm.at[pl.ds(base, CAPB)], pk_v)

        def gather(k, b):
            @pl.when(k < nch)
            def _():
                for t in range(CH64 // 16):
                    gib[b, pl.ds(t * 16, 16)] = (
                        pk_v[pl.ds(k * CH64 + t * 16, 16)] >> 8)
                pltpu.async_copy(hall_hbm.at[gib.at[b]], rowsb.at[b],
                                 gsem[b])

        def accum(k, b):
            @pl.when((k >= 0) & (k < nch))
            def _():
                pltpu.make_async_copy(hall_hbm.at[gib.at[b]],
                                      rowsb.at[b], gsem[b]).wait()
                for t in range(CH64 // 16):
                    dl = pk_v[pl.ds(k * CH64 + t * 16, 16)] & (CAP - 1)
                    for l in range(16):
                        dlx = dl[l]
                        vals = [rowsb[b, t * 16 + l, pl.ds(j * 16, 16)]
                                for j in range(D // 16)]
                        for j in range(D // 16):
                            plsc.addupdate(acc.at[dlx, pl.ds(j * 16, 16)],
                                           vals[j])

        def ring_body(t, _):
            for bb in range(NBUF):
                k = t * NBUF + bb
                accum(k - (NBUF - 1), (bb + 1) % NBUF)
                gather(k, bb)
            return 0

        ntri = (nch + NBUF - 1 + NBUF - 1) // NBUF
        lax.fori_loop(0, ntri, ring_body, 0)

        rem = N - (NBKT - 1) * BROWS

        @pl.when(bkt < NBKT - 1)
        def _():
            pltpu.sync_copy(acc.at[pl.ds(0, BROWS)],
                            out_hbm.at[pl.ds(bkt * BROWS, BROWS)])

        @pl.when(bkt == NBKT - 1)
        def _():
            pltpu.sync_copy(acc.at[pl.ds(0, rem)],
                            out_hbm.at[pl.ds(bkt * BROWS, rem)])

        return 0

    lax.fori_loop(0, 2, bucket_pass, 0)


def kernel(feat, edge_index, etype, basis1, comp1, loop1, bias1,
           basis2, comp2, loop2, bias2, basis3, comp3, loop3, bias3,
           fc_w, fc_b):
    src = edge_index[0]
    dst = edge_index[1]

    packed, cnts = _make_sc_route()(src, dst, etype)

    hall = _tc_first(feat, basis1, comp1)
    agg = _make_sc_scatter()(hall.reshape(N * R, D), packed, cnts)

    h1, hall = _tc_mid(agg, feat, loop1, bias1.reshape(1, D), basis2, comp2)
    agg = _make_sc_scatter()(hall.reshape(N * R, D), packed, cnts)

    h2, hall = _tc_mid(agg, h1, loop2, bias2.reshape(1, D), basis3, comp3)
    agg = _make_sc_scatter()(hall.reshape(N * R, D), packed, cnts)

    return _tc_last(agg, h2, loop3, bias3.reshape(1, D), fc_w,
                    fc_b.reshape(1, D))


# speedup vs baseline: 1.0166x; 1.0166x over previous
import functools

import jax
import jax.numpy as jnp
from jax import lax
from jax.experimental import pallas as pl
from jax.experimental.pallas import tpu as pltpu
from jax.experimental.pallas import tpu_sc as plsc

N = 10000
E = 160000
R = 8
NB = 4
D = 256

NC = 2
NS = 16
NW = NC * NS
EPW = E // NW
NSCAN = EPW // 16 + 1
NBKT = 64
BROWS = 157
ACC_R = 164
TRASH = 160
CAP = 256
CH = 16
CNTW = 64

NBLK = 10
BLK = N // NBLK

_SC_PARAMS = pltpu.CompilerParams(use_tc_tiling_on_sc=False,
                                  needs_layout_passes=False)


def _rel_weight(basis_ref, comp_ref, r):
    def b16(x):
        return x.astype(jnp.bfloat16).astype(jnp.float32)

    w = b16(comp_ref[r, 0]) * b16(basis_ref[0])
    for b in range(1, NB):
        w = w + b16(comp_ref[r, b]) * b16(basis_ref[b])
    return w


def _tc_first_body(x_ref, basis_ref, comp_ref, hall_ref):
    r = pl.program_id(1)
    w = _rel_weight(basis_ref, comp_ref, r)
    hall_ref[...] = jnp.dot(x_ref[...].astype(jnp.bfloat16),
                            w.astype(jnp.bfloat16),
                            preferred_element_type=jnp.float32)


def _tc_first(x, basis, comp):
    return pl.pallas_call(
        _tc_first_body,
        grid=(NBLK, R),
        in_specs=[
            pl.BlockSpec((BLK, D), lambda n, r: (n, 0)),
            pl.BlockSpec((NB, D, D), lambda n, r: (0, 0, 0)),
            pl.BlockSpec(memory_space=pltpu.SMEM),
        ],
        out_specs=pl.BlockSpec((BLK, D), lambda n, r: (n, r)),
        out_shape=jax.ShapeDtypeStruct((N, R * D), jnp.float32),
    )(x, basis, comp)


def _tc_mid_body(agg_ref, x_ref, lw_ref, b_ref, basis_ref, comp_ref,
                 h_ref, hall_ref, hs_ref):
    r = pl.program_id(1)

    @pl.when(r == 0)
    def _():
        t = agg_ref[...] + jnp.dot(x_ref[...].astype(jnp.bfloat16),
                                   lw_ref[...].astype(jnp.bfloat16),
                                   preferred_element_type=jnp.float32)
        t = jnp.maximum(t + b_ref[...], 0.0)
        hs_ref[...] = t
        h_ref[...] = t

    w = _rel_weight(basis_ref, comp_ref, r)
    hall_ref[...] = jnp.dot(hs_ref[...].astype(jnp.bfloat16),
                            w.astype(jnp.bfloat16),
                            preferred_element_type=jnp.float32)


def _tc_mid(agg, x, loop_w, bias, basis, comp):
    return pl.pallas_call(
        _tc_mid_body,
        grid=(NBLK, R),
        in_specs=[
            pl.BlockSpec((BLK, D), lambda n, r: (n, 0)),
            pl.BlockSpec((BLK, D), lambda n, r: (n, 0)),
            pl.BlockSpec((D, D), lambda n, r: (0, 0)),
            pl.BlockSpec((1, D), lambda n, r: (0, 0)),
            pl.BlockSpec((NB, D, D), lambda n, r: (0, 0, 0)),
            pl.BlockSpec(memory_space=pltpu.SMEM),
        ],
        out_specs=[
            pl.BlockSpec((BLK, D), lambda n, r: (n, 0)),
            pl.BlockSpec((BLK, D), lambda n, r: (n, r)),
        ],
        out_shape=[
            jax.ShapeDtypeStruct((N, D), jnp.float32),
            jax.ShapeDtypeStruct((N, R * D), jnp.float32),
        ],
        scratch_shapes=[pltpu.VMEM((BLK, D), jnp.float32)],
    )(agg, x, loop_w, bias, basis, comp)


def _tc_last_body(agg_ref, x_ref, lw_ref, b_ref, fcw_ref, fcb_ref, out_ref):
    t = agg_ref[...] + jnp.dot(x_ref[...].astype(jnp.bfloat16),
                               lw_ref[...].astype(jnp.bfloat16),
                               preferred_element_type=jnp.float32)
    h = jnp.maximum(t + b_ref[...], 0.0)
    t = jnp.dot(h.astype(jnp.bfloat16), fcw_ref[...].astype(jnp.bfloat16),
                preferred_element_type=jnp.float32)
    t = t + fcb_ref[...]
    m = jnp.max(t, axis=1, keepdims=True)
    e = jnp.exp(t - m)
    out_ref[...] = e / jnp.sum(e, axis=1, keepdims=True)


def _tc_last(agg, x, loop_w, bias, fc_w, fc_b):
    return pl.pallas_call(
        _tc_last_body,
        grid=(NBLK,),
        in_specs=[
            pl.BlockSpec((BLK, D), lambda n: (n, 0)),
            pl.BlockSpec((BLK, D), lambda n: (n, 0)),
            pl.BlockSpec((D, D), lambda n: (0, 0)),
            pl.BlockSpec((1, D), lambda n: (0, 0)),
            pl.BlockSpec((D, D), lambda n: (0, 0)),
            pl.BlockSpec((1, D), lambda n: (0, 0)),
        ],
        out_specs=pl.BlockSpec((BLK, D), lambda n: (n, 0)),
        out_shape=jax.ShapeDtypeStruct((N, D), jnp.float32),
    )(agg, x, loop_w, bias, fc_w, fc_b)


EBLK = 5000
CAPB = 4096
CH64 = 64
NBUF = 3
CNTS = 8


def _sc_mesh():
    return plsc.VectorSubcoreMesh(core_axis_name="c", subcore_axis_name="s",
                                  num_cores=NC, num_subcores=NS)


@functools.lru_cache(maxsize=None)
def _make_sc_route():
    return functools.partial(
        pl.kernel,
        mesh=_sc_mesh(),
        compiler_params=_SC_PARAMS,
        out_type=(jax.ShapeDtypeStruct((NBKT * CAPB,), jnp.int32),
                  jax.ShapeDtypeStruct((NBKT * CNTS,), jnp.int32)),
        scratch_types=[
            pltpu.VMEM((EBLK + 16,), jnp.int32),
            pltpu.VMEM((EBLK + 16,), jnp.int32),
            pltpu.VMEM((EBLK + 16,), jnp.int32),
            pltpu.VMEM((CAPB,), jnp.int32),
            pltpu.VMEM((CAPB,), jnp.int32),
            pltpu.VMEM((16,), jnp.int32),
        ],
    )(_sc_route_body)


def _sc_route_body(src_hbm, dst_hbm, et_hbm, packed_hbm, cnt_hbm,
                   src_v, dst_v, et_v, la_v, lb_v, cb_v):
    cid = lax.axis_index("c")
    sid = lax.axis_index("s")
    w = cid * NS + sid
    bka = 2 * w
    iota = lax.iota(jnp.int32, 16)
    nvec = EBLK // 16
    tail = EBLK - nvec * 16

    def block_body(i, offs):
        offa, offb = offs
        base = i * EBLK
        pltpu.sync_copy(src_hbm.at[pl.ds(base, EBLK)],
                        src_v.at[pl.ds(0, EBLK)])
        pltpu.sync_copy(dst_hbm.at[pl.ds(base, EBLK)],
                        dst_v.at[pl.ds(0, EBLK)])
        pltpu.sync_copy(et_hbm.at[pl.ds(base, EBLK)],
                        et_v.at[pl.ds(0, EBLK)])

        def step(sl, valid, offa, offb):
            d = dst_v[sl]
            bk = d // BROWS
            pk = (src_v[sl] * R + et_v[sl]) * CAP + (d - bk * BROWS)
            ma = (bk == bka) & valid
            mb = (bk == bka + 1) & valid
            ca = plsc.all_reduce_population_count(ma)
            cb = plsc.all_reduce_population_count(mb)
            if getattr(ca, "ndim", 0):
                ca, cb = ca[0], cb[0]
            plsc.store_compressed(
                la_v.at[pl.ds(jnp.minimum(offa, CAPB - 16), 16)], pk,
                mask=ma)
            plsc.store_compressed(
                lb_v.at[pl.ds(jnp.minimum(offb, CAPB - 16), 16)], pk,
                mask=mb)
            return offa + ca, offb + cb

        def vec_body(j, offs2):
            return step(pl.ds(j * 16, 16), iota >= 0, *offs2)

        offa, offb = lax.fori_loop(0, nvec, vec_body, (offa, offb))
        offa, offb = step(pl.ds(nvec * 16, 16), iota < tail, offa, offb)
        return offa, offb

    offa, offb = lax.fori_loop(0, E // EBLK, block_body,
                           (jnp.int32(0), jnp.int32(0)))

    sent = jnp.full((16,), TRASH, dtype=jnp.int32)
    for t in range(CH64 // 16):
        la_v[pl.ds(jnp.minimum(offa + t * 16, CAPB - 16), 16)] = sent
        lb_v[pl.ds(jnp.minimum(offb + t * 16, CAPB - 16), 16)] = sent
    cb_v[...] = jnp.where(iota == 0, offa, jnp.where(iota == 8, offb, 0))

    pltpu.sync_copy(la_v, packed_hbm.at[pl.ds(bka * CAPB, CAPB)])
    pltpu.sync_copy(lb_v, packed_hbm.at[pl.ds((bka + 1) * CAPB, CAPB)])
    pltpu.sync_copy(cb_v, cnt_hbm.at[pl.ds(bka * CNTS, 16)])


@functools.lru_cache(maxsize=None)
def _make_sc_scatter():
    return functools.partial(
        pl.kernel,
        mesh=_sc_mesh(),
        compiler_params=_SC_PARAMS,
        out_type=jax.ShapeDtypeStruct((N, D), jnp.float32),
        scratch_types=[
            pltpu.VMEM((NBKT * CNTS + 16,), jnp.int32),
            pltpu.VMEM((CAPB,), jnp.int32),
            pltpu.VMEM((NBUF, CH64), jnp.int32),
            pltpu.VMEM((NBUF, CH64, D), jnp.float32),
            pltpu.VMEM((ACC_R, D), jnp.float32),
        ] + [pltpu.SemaphoreType.DMA] * NBUF,
    )(_sc_scatter_body)


def _sc_scatter_body(hall_hbm, packed_hbm, cnt_hbm, out_hbm,
                     cnt_v, pk_v, gib, rowsb, acc, *gsem):
    cid = lax.axis_index("c")
    sid = lax.axis_index("s")
    w = cid * NS + sid

    pltpu.sync_copy(cnt_hbm, cnt_v.at[pl.ds(0, NBKT * CNTS)])
    zf = jnp.zeros((16,), jnp.float32)

    def bucket_pass(g, _):
        bkt = 2 * w + g

        def zero_body(i, _):
            for j in range(D // 16):
                acc[i, pl.ds(j * 16, 16)] = zf
            return 0

        lax.fori_loop(0, ACC_R, zero_body, 0)

        tot = cnt_v[pl.ds(bkt * CNTS, 16)][0]
        nch = (tot + CH64 - 1) // CH64
        base = bkt * CAPB
        pltpu.sync_copy(packed_hbm.at[pl.ds(base, CAPB)], pk_v)

        def gather(k, b):
            @pl.when(k < nch)
            def _():
                for t in range(CH64 // 16):
                    gib[b, pl.ds(t * 16, 16)] = (
                        pk_v[pl.ds(k * CH64 + t * 16, 16)] >> 8)
                pltpu.async_copy(hall_hbm.at[gib.at[b]], rowsb.at[b],
                                 gsem[b])

        def accum(k, b):
            @pl.when((k >= 0) & (k < nch))
            def _():
                pltpu.make_async_copy(hall_hbm.at[gib.at[b]],
                                      rowsb.at[b], gsem[b]).wait()
                for t in range(CH64 // 16):
                    dl = pk_v[pl.ds(k * CH64 + t * 16, 16)] & (CAP - 1)
                    for l in range(16):
                        dlx = dl[l]
                        vals = [rowsb[b, t * 16 + l, pl.ds(j * 16, 16)]
                                for j in range(D // 16)]
                        for j in range(D // 16):
                            plsc.addupdate(acc.at[dlx, pl.ds(j * 16, 16)],
                                           vals[j])

        def ring_body(t, _):
            for bb in range(NBUF):
                k = t * NBUF + bb
                accum(k - (NBUF - 1), (bb + 1) % NBUF)
                gather(k, bb)
            return 0

        ntri = (nch + NBUF - 1 + NBUF - 1) // NBUF
        lax.fori_loop(0, ntri, ring_body, 0)

        rem = N - (NBKT - 1) * BROWS

        @pl.when(bkt < NBKT - 1)
        def _():
            pltpu.sync_copy(acc.at[pl.ds(0, BROWS)],
                            out_hbm.at[pl.ds(bkt * BROWS, BROWS)])

        @pl.when(bkt == NBKT - 1)
        def _():
            pltpu.sync_copy(acc.at[pl.ds(0, rem)],
                            out_hbm.at[pl.ds(bkt * BROWS, rem)])

        return 0

    lax.fori_loop(0, 2, bucket_pass, 0)


def kernel(feat, edge_index, etype, basis1, comp1, loop1, bias1,
           basis2, comp2, loop2, bias2, basis3, comp3, loop3, bias3,
           fc_w, fc_b):
    src = edge_index[0]
    dst = edge_index[1]

    packed, cnts = _make_sc_route()(src, dst, etype)

    hall = _tc_first(feat, basis1, comp1)
    agg = _make_sc_scatter()(hall.reshape(N * R, D), packed, cnts)

    h1, hall = _tc_mid(agg, feat, loop1, bias1.reshape(1, D), basis2, comp2)
    agg = _make_sc_scatter()(hall.reshape(N * R, D), packed, cnts)

    h2, hall = _tc_mid(agg, h1, loop2, bias2.reshape(1, D), basis3, comp3)
    agg = _make_sc_scatter()(hall.reshape(N * R, D), packed, cnts)

    return _tc_last(agg, h2, loop3, bias3.reshape(1, D), fc_w,
                    fc_b.reshape(1, D))
